# Initial kernel scaffold; baseline (speedup 1.0000x reference)
import jax
import jax.numpy as jnp
from jax.experimental import pallas as pl


def kernel(x, W, b):
    raise NotImplementedError("write your pallas kernel here")


# speedup vs baseline: 1.2900x; 1.2900x over previous
import functools

import jax
import jax.numpy as jnp
from jax.experimental import pallas as pl
from jax.experimental.pallas import tpu as pltpu

_TILE = 512


def _router_body(x_ref, wt_ref, b_ref, gate_ref, v_ref, i_ref):
    logits = jnp.dot(x_ref[...], wt_ref[...],
                     preferred_element_type=jnp.float32) + b_ref[...]
    m = jnp.max(logits, axis=-1, keepdims=True)
    e = jnp.exp(logits - m)
    s = jnp.sum(e, axis=-1, keepdims=True)
    g = e / s
    gate_ref[...] = g
    E = g.shape[-1]
    ids = jax.lax.broadcasted_iota(jnp.int32, g.shape, 1)
    m1 = jnp.max(g, axis=-1, keepdims=True)
    i1 = jnp.min(jnp.where(g == m1, ids, E), axis=-1, keepdims=True)
    g2 = jnp.where(ids == i1, -jnp.inf, g)
    m2 = jnp.max(g2, axis=-1, keepdims=True)
    i2 = jnp.min(jnp.where(g2 == m2, ids, E), axis=-1, keepdims=True)
    v_ref[...] = jnp.concatenate([m1, m2], axis=-1)
    i_ref[...] = jnp.concatenate([i1, i2], axis=-1)


@jax.jit
def kernel(x, W, b):
    B, S, D = x.shape
    E = W.shape[0]
    N = B * S
    xf = x.reshape(N, D)
    wt = W.T
    b2 = b.reshape(1, E)
    grid = (N // _TILE,)
    gate, v, i = pl.pallas_call(
        _router_body,
        grid=grid,
        in_specs=[
            pl.BlockSpec((_TILE, D), lambda i: (i, 0)),
            pl.BlockSpec((D, E), lambda i: (0, 0)),
            pl.BlockSpec((1, E), lambda i: (0, 0)),
        ],
        out_specs=[
            pl.BlockSpec((_TILE, E), lambda i: (i, 0)),
            pl.BlockSpec((_TILE, 2), lambda i: (i, 0)),
            pl.BlockSpec((_TILE, 2), lambda i: (i, 0)),
        ],
        out_shape=[
            jax.ShapeDtypeStruct((N, E), jnp.float32),
            jax.ShapeDtypeStruct((N, 2), jnp.float32),
            jax.ShapeDtypeStruct((N, 2), jnp.int32),
        ],
    )(xf, wt, b2)
    return (v.reshape(B, S, 2),
            i.reshape(B, S, 2).astype(jnp.int64),
            gate.reshape(B, S, E))


# speedup vs baseline: 1.2913x; 1.0010x over previous
import functools

import jax
import jax.numpy as jnp
from jax.experimental import pallas as pl
from jax.experimental.pallas import tpu as pltpu

_TILE = 512


def _router_body(xa_ref, xb_ref, wt_ref, b_ref, gate_ref, v_ref, i_ref):
    h = wt_ref.shape[0] // 2
    logits = (jnp.dot(xa_ref[...], wt_ref[:h],
                      preferred_element_type=jnp.float32)
              + jnp.dot(xb_ref[...], wt_ref[h:],
                        preferred_element_type=jnp.float32)
              + b_ref[...])
    m = jnp.max(logits, axis=-1, keepdims=True)
    e = jnp.exp(logits - m)
    s = jnp.sum(e, axis=-1, keepdims=True)
    g = e / s
    gate_ref[...] = g
    E = g.shape[-1]
    ids = jax.lax.broadcasted_iota(jnp.int32, g.shape, 1)
    m1 = jnp.max(g, axis=-1, keepdims=True)
    i1 = jnp.min(jnp.where(g == m1, ids, E), axis=-1, keepdims=True)
    g2 = jnp.where(ids == i1, -jnp.inf, g)
    m2 = jnp.max(g2, axis=-1, keepdims=True)
    i2 = jnp.min(jnp.where(g2 == m2, ids, E), axis=-1, keepdims=True)
    v_ref[...] = jnp.concatenate([m1, m2], axis=-1)
    i_ref[...] = jnp.concatenate([i1, i2], axis=-1)


@jax.jit
def kernel(x, W, b):
    B, S, D = x.shape
    E = W.shape[0]
    N = B * S
    xf = x.reshape(N, D)
    wt = W.T
    b2 = b.reshape(1, E)
    grid = (N // _TILE,)
    gate, v, i = pl.pallas_call(
        _router_body,
        grid=grid,
        in_specs=[
            pl.BlockSpec((_TILE, D // 2), lambda i: (i, 0)),
            pl.BlockSpec((_TILE, D // 2), lambda i: (i, 1)),
            pl.BlockSpec((D, E), lambda i: (0, 0)),
            pl.BlockSpec((1, E), lambda i: (0, 0)),
        ],
        out_specs=[
            pl.BlockSpec((_TILE, E), lambda i: (i, 0)),
            pl.BlockSpec((_TILE, 2), lambda i: (i, 0)),
            pl.BlockSpec((_TILE, 2), lambda i: (i, 0)),
        ],
        out_shape=[
            jax.ShapeDtypeStruct((N, E), jnp.float32),
            jax.ShapeDtypeStruct((N, 2), jnp.float32),
            jax.ShapeDtypeStruct((N, 2), jnp.int32),
        ],
    )(xf, xf, wt, b2)
    return (v.reshape(B, S, 2),
            i.reshape(B, S, 2).astype(jnp.int64),
            gate.reshape(B, S, E))
